# baseline (speedup 1.0000x reference)
import functools

import jax
import jax.numpy as jnp
from jax import lax
from jax.experimental import pallas as pl
from jax.experimental.pallas import tpu as pltpu
from jax.experimental.pallas import tpu_sc as plsc

_B, _L = 4096, 200
_D = 32
_N = _B * _L

_NC, _NS = 2, 16
_NW = _NC * _NS
_PW = _N // _NW
_CHUNK = 3200
_NCHUNK = _PW // _CHUNK


def _area_body(a_ref, w_ref, b_ref, o_ref):
    a = a_ref[...]
    w = w_ref[...]
    b = b_ref[...]
    y = a[:, :, 0:1] * w[0:1, :][None] + a[:, :, 1:2] * w[1:2, :][None]
    o_ref[...] = jnp.maximum(y + b[None], 0.0)


_AREA_BB = 32

_area_call = pl.pallas_call(
    _area_body,
    out_shape=jax.ShapeDtypeStruct((_B, _L, _D), jnp.float32),
    grid=(_B // _AREA_BB,),
    in_specs=[
        pl.BlockSpec((_AREA_BB, _L, 2), lambda i: (i, 0, 0)),
        pl.BlockSpec((2, _D), lambda i: (0, 0)),
        pl.BlockSpec((1, _D), lambda i: (0, 0)),
    ],
    out_specs=pl.BlockSpec((_AREA_BB, _L, _D), lambda i: (i, 0, 0)),
)


_RW = _B // _NW
_CROWS = _CHUNK // _L


def _gather_body(shot_hbm, player_hbm, stab_hbm, ptab_hbm, out_s, out_p,
                 idx_v, rows_v, sem):
    wid = lax.axis_index("s") * _NC + lax.axis_index("c")
    base = wid * _RW
    for tab, idx_hbm, out in ((stab_hbm, shot_hbm, out_s),
                              (ptab_hbm, player_hbm, out_p)):
        for k in range(_RW // _CROWS):
            off = base + k * _CROWS
            pltpu.sync_copy(idx_hbm.at[pl.ds(off, _CROWS)], idx_v)
            copies = [pltpu.async_copy(tab.at[idx_v.at[r]], rows_v.at[r], sem)
                      for r in range(_CROWS)]
            for c in copies:
                c.wait()
            pltpu.sync_copy(rows_v, out.at[pl.ds(off, _CROWS)])


_gather_call = functools.partial(
    pl.kernel,
    out_type=(jax.ShapeDtypeStruct((_B, _L, _D), jnp.float32),
              jax.ShapeDtypeStruct((_B, _L, _D), jnp.float32)),
    mesh=plsc.VectorSubcoreMesh(core_axis_name="c", subcore_axis_name="s",
                                num_cores=_NC, num_subcores=_NS),
    scratch_types=[
        pltpu.VMEM((_CROWS, _L), jnp.int32),
        pltpu.VMEM((_CROWS, _L, _D), jnp.float32),
        pltpu.SemaphoreType.DMA,
    ],
    compiler_params=pltpu.CompilerParams(use_tc_tiling_on_sc=False),
)(_gather_body)


def kernel(area, shot, player, W_area, b_area, shot_table, player_table):
    shot_i = shot.astype(jnp.int32)
    player_i = player.astype(jnp.int32)
    emb_area = _area_call(area, W_area, b_area.reshape(1, _D))
    emb_shot, emb_player = _gather_call(shot_i, player_i,
                                        shot_table, player_table)
    return (emb_area, emb_shot, emb_player)


# speedup vs baseline: 1.5052x; 1.5052x over previous
import functools

import jax
import jax.numpy as jnp
from jax import lax
from jax.experimental import pallas as pl
from jax.experimental.pallas import tpu as pltpu
from jax.experimental.pallas import tpu_sc as plsc

_B, _L = 4096, 200
_D = 32
_N = _B * _L

_NC, _NS = 2, 16
_NW = _NC * _NS
_PW = _N // _NW
_CHUNK = 3200
_NCHUNK = _PW // _CHUNK


def _area_body(x0_ref, x1_ref, wt_ref, bt_ref, o_ref):
    x0 = x0_ref[...]
    x1 = x1_ref[...]
    wt = wt_ref[...]
    bt = bt_ref[...]
    w0 = wt[:, 0:1][None]
    w1 = wt[:, 1:2][None]
    y = x0[:, None, :] * w0 + x1[:, None, :] * w1 + bt[None]
    o_ref[...] = jnp.maximum(y, 0.0)


_AREA_BB = 64

_area_call = pl.pallas_call(
    _area_body,
    out_shape=jax.ShapeDtypeStruct((_B, _D, _L), jnp.float32),
    grid=(_B // _AREA_BB,),
    in_specs=[
        pl.BlockSpec((_AREA_BB, _L), lambda i: (i, 0)),
        pl.BlockSpec((_AREA_BB, _L), lambda i: (i, 0)),
        pl.BlockSpec((_D, 2), lambda i: (0, 0)),
        pl.BlockSpec((_D, 1), lambda i: (0, 0)),
    ],
    out_specs=pl.BlockSpec((_AREA_BB, _D, _L), lambda i: (i, 0, 0)),
)


_RW = _B // _NW
_CROWS = _CHUNK // _L


def _gather_body(shot_hbm, player_hbm, stab_hbm, ptab_hbm, out_s, out_p,
                 idx_v, rows_v, sem):
    wid = lax.axis_index("s") * _NC + lax.axis_index("c")
    base = wid * _RW
    for tab, idx_hbm, out in ((stab_hbm, shot_hbm, out_s),
                              (ptab_hbm, player_hbm, out_p)):
        for k in range(_RW // _CROWS):
            off = base + k * _CROWS
            pltpu.sync_copy(idx_hbm.at[pl.ds(off, _CROWS)], idx_v)
            copies = [pltpu.async_copy(tab.at[idx_v.at[r]], rows_v.at[r], sem)
                      for r in range(_CROWS)]
            for c in copies:
                c.wait()
            pltpu.sync_copy(rows_v, out.at[pl.ds(off, _CROWS)])


_gather_call = functools.partial(
    pl.kernel,
    out_type=(jax.ShapeDtypeStruct((_B, _L, _D), jnp.float32),
              jax.ShapeDtypeStruct((_B, _L, _D), jnp.float32)),
    mesh=plsc.VectorSubcoreMesh(core_axis_name="c", subcore_axis_name="s",
                                num_cores=_NC, num_subcores=_NS),
    scratch_types=[
        pltpu.VMEM((_CROWS, _L), jnp.int32),
        pltpu.VMEM((_CROWS, _L, _D), jnp.float32),
        pltpu.SemaphoreType.DMA,
    ],
    compiler_params=pltpu.CompilerParams(use_tc_tiling_on_sc=False),
)(_gather_body)


def kernel(area, shot, player, W_area, b_area, shot_table, player_table):
    shot_i = shot.astype(jnp.int32)
    player_i = player.astype(jnp.int32)
    emb_area_t = _area_call(area[:, :, 0], area[:, :, 1],
                            W_area.T, b_area[:, None])
    emb_area = emb_area_t.transpose(0, 2, 1)
    emb_shot, emb_player = _gather_call(shot_i, player_i,
                                        shot_table, player_table)
    return (emb_area, emb_shot, emb_player)


# speedup vs baseline: 1.6436x; 1.0919x over previous
import functools

import jax
import jax.numpy as jnp
from jax import lax
from jax.experimental import pallas as pl
from jax.experimental.pallas import tpu as pltpu
from jax.experimental.pallas import tpu_sc as plsc

_B, _L = 4096, 200
_D = 32
_N = _B * _L

_NC, _NS = 2, 16
_NW = _NC * _NS
_RW = _B // _NW
_CROWS = 16


def _area_body(x0_ref, x1_ref, wt_ref, bt_ref, o_ref):
    x0 = x0_ref[...]
    x1 = x1_ref[...]
    wt = wt_ref[...]
    bt = bt_ref[...]
    w0 = wt[:, 0:1][None]
    w1 = wt[:, 1:2][None]
    y = x0[:, None, :] * w0 + x1[:, None, :] * w1 + bt[None]
    o_ref[...] = jnp.maximum(y, 0.0)


_AREA_LB = 8

_area_call = pl.pallas_call(
    _area_body,
    out_shape=jax.ShapeDtypeStruct((_L, _D, _B), jnp.float32),
    grid=(_L // _AREA_LB,),
    in_specs=[
        pl.BlockSpec((_AREA_LB, _B), lambda i: (i, 0)),
        pl.BlockSpec((_AREA_LB, _B), lambda i: (i, 0)),
        pl.BlockSpec((_D, 2), lambda i: (0, 0)),
        pl.BlockSpec((_D, 1), lambda i: (0, 0)),
    ],
    out_specs=pl.BlockSpec((_AREA_LB, _D, _B), lambda i: (i, 0, 0)),
)


def _gather_body(shot_hbm, player_hbm, stab_hbm, ptab_hbm, out_s, out_p,
                 idx_v, rows_v, sem):
    wid = lax.axis_index("s") * _NC + lax.axis_index("c")
    base = wid * _RW
    for tab, idx_hbm, out in ((stab_hbm, shot_hbm, out_s),
                              (ptab_hbm, player_hbm, out_p)):
        for k in range(_RW // _CROWS):
            off = base + k * _CROWS
            pltpu.sync_copy(idx_hbm.at[pl.ds(off, _CROWS)], idx_v)
            copies = [pltpu.async_copy(tab.at[idx_v.at[r]], rows_v.at[r], sem)
                      for r in range(_CROWS)]
            for c in copies:
                c.wait()
            pltpu.sync_copy(rows_v, out.at[pl.ds(off, _CROWS)])


_gather_call = functools.partial(
    pl.kernel,
    out_type=(jax.ShapeDtypeStruct((_B, _L, _D), jnp.float32),
              jax.ShapeDtypeStruct((_B, _L, _D), jnp.float32)),
    mesh=plsc.VectorSubcoreMesh(core_axis_name="c", subcore_axis_name="s",
                                num_cores=_NC, num_subcores=_NS),
    scratch_types=[
        pltpu.VMEM((_CROWS, _L), jnp.int32),
        pltpu.VMEM((_CROWS, _L, _D), jnp.float32),
        pltpu.SemaphoreType.DMA,
    ],
    compiler_params=pltpu.CompilerParams(use_tc_tiling_on_sc=False),
)(_gather_body)


def kernel(area, shot, player, W_area, b_area, shot_table, player_table):
    shot_i = shot.astype(jnp.int32)
    player_i = player.astype(jnp.int32)
    emb_area_t = _area_call(area[:, :, 0].T, area[:, :, 1].T,
                            W_area.T, b_area[:, None])
    emb_area = emb_area_t.transpose(2, 0, 1)
    emb_shot, emb_player = _gather_call(shot_i, player_i,
                                        shot_table, player_table)
    return (emb_area, emb_shot, emb_player)
